# baseline (speedup 1.0000x reference)
import jax
import jax.numpy as jnp
from jax import lax
from jax.experimental import pallas as pl
from jax.experimental.pallas import tpu as pltpu
from jax.experimental.pallas import tpu_sc as plsc

VOCAB = 1000000
ROWS = 65536
N_CHUNKS = 8
CHUNK = 16
B = 16384

NC = 2
NS = 16
NW = NC * NS
TPW = B // NW
RPW = TPW * N_CHUNKS
RPS = TPW


def _body(x_hbm, t0_hbm, t1_hbm, w_hbm, h0_hbm, h1_hbm, h2_hbm, out_hbm,
          x_v, xe_v, g0_v, g1_v, g2_v, w0_v, w1_v, p0a, p1a, p0b, p1b,
          out_v, sem, sem_b):
    c = lax.axis_index("c")
    s = lax.axis_index("s")
    wid = s * NC + c
    tok_base = wid * TPW

    pltpu.sync_copy(x_hbm.at[pl.ds(tok_base, TPW)], x_v)

    def xe_body(i, _):
        cc = lax.shift_right_logical(i, 5)
        bo = jnp.bitwise_and(i, 31) * 16
        xv = x_v[pl.ds(bo, 16)]
        xe_v[pl.ds(i * 16, 16)] = (
            lax.shift_left(lax.shift_right_logical(xv, 7), 10)
            + cc * 128 + jnp.bitwise_and(xv, 127))
        return 0

    lax.fori_loop(0, RPW // 16, xe_body, 0)

    cp0 = pltpu.async_copy(h0_hbm.at[xe_v], g0_v, sem)
    cp1 = pltpu.async_copy(h1_hbm.at[xe_v], g1_v, sem)
    cp2 = pltpu.async_copy(h2_hbm.at[xe_v], g2_v, sem)
    cp0.wait()
    cp1.wait()
    cp2.wait()

    def idx_body(i, _):
        cc = lax.shift_right_logical(i, 5)
        sl = pl.ds(i * 16, 16)
        g0_v[sl] = g0_v[sl] * N_CHUNKS + cc
        g1_v[sl] = g1_v[sl] * N_CHUNKS + cc
        h2v = g2_v[sl]
        base = (cc * (2 * ROWS)
                + lax.shift_left(lax.shift_right_logical(h2v, 7), 8)
                + jnp.bitwise_and(h2v, 127))
        xe_v[sl] = base
        g2_v[sl] = base + 128
        return 0

    lax.fori_loop(0, RPW // 16, idx_body, 0)

    cpw0 = pltpu.async_copy(w_hbm.at[xe_v], w0_v, sem)
    cpw1 = pltpu.async_copy(w_hbm.at[g2_v], w1_v, sem)
    cpw0.wait()
    cpw1.wait()

    bufs = ((p0a, p1a), (p0b, p1b))
    sems = (sem, sem_b)

    def fire(sb):
        bb = sb & 1
        sl = pl.ds(sb * RPS, RPS)
        return (pltpu.async_copy(t0_hbm.at[g0_v.at[sl]], bufs[bb][0],
                                 sems[bb]),
                pltpu.async_copy(t1_hbm.at[g1_v.at[sl]], bufs[bb][1],
                                 sems[bb]))

    cur = fire(0)
    for sb in range(N_CHUNKS):
        nxt = fire(sb + 1) if sb + 1 < N_CHUNKS else None
        cur[0].wait()
        cur[1].wait()
        p0_v, p1_v = bufs[sb & 1]

        def row_body(g, _):
            for u in range(4):
                m = g * 4 + u
                qv = jnp.full((16,), sb * RPS + m, jnp.int32)
                w0 = plsc.load_gather(w0_v, [qv])
                w1 = plsc.load_gather(w1_v, [qv])
                out_v[m, pl.ds(sb * CHUNK, CHUNK)] = (
                    p0_v[m, :] * w0 + p1_v[m, :] * w1)
            return 0

        lax.fori_loop(0, RPS // 4, row_body, 0)
        cur = nxt

    pltpu.sync_copy(out_v, out_hbm.at[pl.ds(tok_base, TPW), :])


@jax.jit
def _call(x, t0, t1, w, h0f, h1f, h2f):
    mesh = plsc.VectorSubcoreMesh(core_axis_name="c", subcore_axis_name="s")
    run = pl.kernel(
        _body,
        out_type=jax.ShapeDtypeStruct((B, N_CHUNKS * CHUNK), jnp.float32),
        mesh=mesh,
        compiler_params=pltpu.CompilerParams(use_tc_tiling_on_sc=False,
                                             needs_layout_passes=False),
        scratch_types=[
            pltpu.VMEM((TPW,), jnp.int32),
            pltpu.VMEM((RPW,), jnp.int32),
            pltpu.VMEM((RPW,), jnp.int32),
            pltpu.VMEM((RPW,), jnp.int32),
            pltpu.VMEM((RPW,), jnp.int32),
            pltpu.VMEM((RPW,), jnp.float32),
            pltpu.VMEM((RPW,), jnp.float32),
            pltpu.VMEM((RPS, CHUNK), jnp.float32),
            pltpu.VMEM((RPS, CHUNK), jnp.float32),
            pltpu.VMEM((RPS, CHUNK), jnp.float32),
            pltpu.VMEM((RPS, CHUNK), jnp.float32),
            pltpu.VMEM((TPW, 128), jnp.float32),
            pltpu.SemaphoreType.DMA,
            pltpu.SemaphoreType.DMA,
        ],
    )
    return run(x, t0, t1, w, h0f, h1f, h2f)


HB = 65536
TILES = (VOCAB + 127) // 128
PTILES = 7816
HPAD = PTILES * 128


def _lin_body(i0_ref, i1_ref, i2_ref, o0_ref, o1_ref, o2_ref):
    for i_ref, o_ref in ((i0_ref, o0_ref), (i1_ref, o1_ref),
                         (i2_ref, o2_ref)):
        o_ref[...] = jnp.transpose(
            i_ref[...].reshape(N_CHUNKS, HB // 128, 128), (1, 0, 2))


def _w_body(w_ref, o_ref):
    t = w_ref[...].reshape(N_CHUNKS, 2, ROWS // 128, 128)
    o_ref[...] = jnp.transpose(t, (0, 2, 1, 3)).reshape(
        N_CHUNKS, ROWS // 64, 128)


def _w_linearize(wp):
    return pl.pallas_call(
        _w_body,
        in_specs=[pl.BlockSpec((N_CHUNKS, 2, ROWS), lambda: (0, 0, 0))],
        out_specs=pl.BlockSpec((N_CHUNKS, ROWS // 64, 128),
                               lambda: (0, 0, 0)),
        out_shape=jax.ShapeDtypeStruct((N_CHUNKS, ROWS // 64, 128),
                                       jnp.float32),
    )(wp)


WT = 4096


def _tab_body(t0_ref, t1_ref, o0_ref, o1_ref):
    o0_ref[...] = jnp.transpose(t0_ref[...].reshape(128, WT))
    o1_ref[...] = jnp.transpose(t1_ref[...].reshape(128, WT))


def _tab_linearize(t0p, t1p):
    spec_i = pl.BlockSpec((N_CHUNKS, CHUNK, WT), lambda rb: (0, 0, rb))
    spec_o = pl.BlockSpec((WT, 128), lambda rb: (rb, 0))
    out_t = jax.ShapeDtypeStruct((ROWS, 128), jnp.float32)
    return pl.pallas_call(
        _tab_body,
        grid=(ROWS // WT,),
        in_specs=[spec_i, spec_i],
        out_specs=[spec_o, spec_o],
        out_shape=[out_t, out_t],
    )(t0p, t1p)


def _linearize(h0t, h1t, h2t):
    spec_i = pl.BlockSpec((N_CHUNKS, HB), lambda xb: (0, xb))
    spec_o = pl.BlockSpec((HB // 128, N_CHUNKS, 128), lambda xb: (xb, 0, 0))
    out_t = jax.ShapeDtypeStruct((PTILES, N_CHUNKS, 128), jnp.int32)
    return pl.pallas_call(
        _lin_body,
        grid=((VOCAB + HB - 1) // HB,),
        in_specs=[spec_i, spec_i, spec_i],
        out_specs=[spec_o, spec_o, spec_o],
        out_shape=[out_t, out_t, out_t],
    )(h0t, h1t, h2t)


def kernel(x, table0, table1, weights, h0, h1, h2):
    t0w, t1w = _tab_linearize(jnp.transpose(table0, (1, 2, 0)),
                              jnp.transpose(table1, (1, 2, 0)))
    t0 = t0w.reshape(ROWS * N_CHUNKS, CHUNK)
    t1 = t1w.reshape(ROWS * N_CHUNKS, CHUNK)
    w = _w_linearize(jnp.transpose(weights, (1, 2, 0))).reshape(
        ROWS * N_CHUNKS * 2)
    h0f, h1f, h2f = (h.reshape(PTILES * N_CHUNKS * 128)
                     for h in _linearize(jnp.transpose(h0), jnp.transpose(h1),
                                         jnp.transpose(h2)))
    return _call(x, t0, t1, w, h0f, h1f, h2f)


# speedup vs baseline: 1.0633x; 1.0633x over previous
import jax
import jax.numpy as jnp
from jax import lax
from jax.experimental import pallas as pl
from jax.experimental.pallas import tpu as pltpu
from jax.experimental.pallas import tpu_sc as plsc

VOCAB = 1000000
ROWS = 65536
N_CHUNKS = 8
CHUNK = 16
B = 16384

NC = 2
NS = 16
NW = NC * NS
TPW = B // NW
RPW = TPW * N_CHUNKS
RPS = TPW


def _idx_body(x_hbm, h0_hbm, h1_hbm, h2_hbm, i0_hbm, i1_hbm, iw0_hbm,
              iw1_hbm, x_v, xe_v, g0_v, g1_v, g2_v, sem):
    c = lax.axis_index("c")
    s = lax.axis_index("s")
    wid = s * NC + c
    tok_base = wid * TPW

    pltpu.sync_copy(x_hbm.at[pl.ds(tok_base, TPW)], x_v)

    def xe_body(i, _):
        cc = lax.shift_right_logical(i, 5)
        bo = jnp.bitwise_and(i, 31) * 16
        xv = x_v[pl.ds(bo, 16)]
        xe_v[pl.ds(i * 16, 16)] = (
            lax.shift_left(lax.shift_right_logical(xv, 7), 10)
            + cc * 128 + jnp.bitwise_and(xv, 127))
        return 0

    lax.fori_loop(0, RPW // 16, xe_body, 0)

    cp0 = pltpu.async_copy(h0_hbm.at[xe_v], g0_v, sem)
    cp1 = pltpu.async_copy(h1_hbm.at[xe_v], g1_v, sem)
    cp2 = pltpu.async_copy(h2_hbm.at[xe_v], g2_v, sem)
    cp0.wait()
    cp1.wait()
    cp2.wait()

    def idx_body(i, _):
        cc = lax.shift_right_logical(i, 5)
        sl = pl.ds(i * 16, 16)
        g0_v[sl] = g0_v[sl] * N_CHUNKS + cc
        g1_v[sl] = g1_v[sl] * N_CHUNKS + cc
        h2v = g2_v[sl]
        base = (cc * (2 * ROWS)
                + lax.shift_left(lax.shift_right_logical(h2v, 7), 8)
                + jnp.bitwise_and(h2v, 127))
        xe_v[sl] = base
        g2_v[sl] = base + 128
        return 0

    lax.fori_loop(0, RPW // 16, idx_body, 0)

    row_base = pl.ds(wid * RPW, RPW)
    pltpu.sync_copy(g0_v, i0_hbm.at[row_base])
    pltpu.sync_copy(g1_v, i1_hbm.at[row_base])
    pltpu.sync_copy(xe_v, iw0_hbm.at[row_base])
    pltpu.sync_copy(g2_v, iw1_hbm.at[row_base])


def _gather_body(t0_hbm, t1_hbm, w_hbm, i0_hbm, i1_hbm, iw0_hbm, iw1_hbm,
                 out_hbm, g0_v, g1_v, iw0_v, iw1_v, w0_v, w1_v, p0a, p1a,
                 p0b, p1b, out_v, sem, sem_b):
    c = lax.axis_index("c")
    s = lax.axis_index("s")
    wid = s * NC + c
    tok_base = wid * TPW
    row_base = pl.ds(wid * RPW, RPW)

    pltpu.sync_copy(i0_hbm.at[row_base], g0_v)
    pltpu.sync_copy(i1_hbm.at[row_base], g1_v)
    pltpu.sync_copy(iw0_hbm.at[row_base], iw0_v)
    pltpu.sync_copy(iw1_hbm.at[row_base], iw1_v)

    cpw0 = pltpu.async_copy(w_hbm.at[iw0_v], w0_v, sem)
    cpw1 = pltpu.async_copy(w_hbm.at[iw1_v], w1_v, sem)
    cpw0.wait()
    cpw1.wait()

    bufs = ((p0a, p1a), (p0b, p1b))
    sems = (sem, sem_b)

    def fire(sb):
        bb = sb & 1
        sl = pl.ds(sb * RPS, RPS)
        return (pltpu.async_copy(t0_hbm.at[g0_v.at[sl]], bufs[bb][0],
                                 sems[bb]),
                pltpu.async_copy(t1_hbm.at[g1_v.at[sl]], bufs[bb][1],
                                 sems[bb]))

    cur = fire(0)
    for sb in range(N_CHUNKS):
        nxt = fire(sb + 1) if sb + 1 < N_CHUNKS else None
        cur[0].wait()
        cur[1].wait()
        p0_v, p1_v = bufs[sb & 1]

        def row_body(m, _):
            qv = jnp.full((16,), sb * RPS + m, jnp.int32)
            w0 = plsc.load_gather(w0_v, [qv])
            w1 = plsc.load_gather(w1_v, [qv])
            out_v[m, pl.ds(sb * CHUNK, CHUNK)] = (
                p0_v[m, :] * w0 + p1_v[m, :] * w1)
            return 0

        lax.fori_loop(0, RPS, row_body, 0)
        cur = nxt

    pltpu.sync_copy(out_v, out_hbm.at[pl.ds(tok_base, TPW), :])


@jax.jit
def _call_a(x, h0f, h1f, h2f):
    mesh = plsc.VectorSubcoreMesh(core_axis_name="c", subcore_axis_name="s")
    it = jax.ShapeDtypeStruct((B * N_CHUNKS,), jnp.int32)
    run = pl.kernel(
        _idx_body,
        out_type=(it, it, it, it),
        mesh=mesh,
        compiler_params=pltpu.CompilerParams(use_tc_tiling_on_sc=False,
                                             needs_layout_passes=False),
        scratch_types=[
            pltpu.VMEM((TPW,), jnp.int32),
            pltpu.VMEM((RPW,), jnp.int32),
            pltpu.VMEM((RPW,), jnp.int32),
            pltpu.VMEM((RPW,), jnp.int32),
            pltpu.VMEM((RPW,), jnp.int32),
            pltpu.SemaphoreType.DMA,
        ],
    )
    return run(x, h0f, h1f, h2f)


@jax.jit
def _call_b(t0, t1, w, i0, i1, iw0, iw1):
    mesh = plsc.VectorSubcoreMesh(core_axis_name="c", subcore_axis_name="s")
    run = pl.kernel(
        _gather_body,
        out_type=jax.ShapeDtypeStruct((B, N_CHUNKS * CHUNK), jnp.float32),
        mesh=mesh,
        compiler_params=pltpu.CompilerParams(use_tc_tiling_on_sc=False,
                                             needs_layout_passes=False),
        scratch_types=[
            pltpu.VMEM((RPW,), jnp.int32),
            pltpu.VMEM((RPW,), jnp.int32),
            pltpu.VMEM((RPW,), jnp.int32),
            pltpu.VMEM((RPW,), jnp.int32),
            pltpu.VMEM((RPW,), jnp.float32),
            pltpu.VMEM((RPW,), jnp.float32),
            pltpu.VMEM((RPS, CHUNK), jnp.float32),
            pltpu.VMEM((RPS, CHUNK), jnp.float32),
            pltpu.VMEM((RPS, CHUNK), jnp.float32),
            pltpu.VMEM((RPS, CHUNK), jnp.float32),
            pltpu.VMEM((TPW, 128), jnp.float32),
            pltpu.SemaphoreType.DMA,
            pltpu.SemaphoreType.DMA,
        ],
    )
    return run(t0, t1, w, i0, i1, iw0, iw1)


HB = 65536
TILES = (VOCAB + 127) // 128
PTILES = 7816
HPAD = PTILES * 128


def _lin_body(i0_ref, i1_ref, i2_ref, o0_ref, o1_ref, o2_ref):
    for i_ref, o_ref in ((i0_ref, o0_ref), (i1_ref, o1_ref),
                         (i2_ref, o2_ref)):
        o_ref[...] = jnp.transpose(
            i_ref[...].reshape(N_CHUNKS, HB // 128, 128), (1, 0, 2))


def _w_body(w_ref, o_ref):
    t = w_ref[...].reshape(N_CHUNKS, 2, ROWS // 128, 128)
    o_ref[...] = jnp.transpose(t, (0, 2, 1, 3)).reshape(
        N_CHUNKS, ROWS // 64, 128)


def _w_linearize(wp):
    return pl.pallas_call(
        _w_body,
        in_specs=[pl.BlockSpec((N_CHUNKS, 2, ROWS), lambda: (0, 0, 0))],
        out_specs=pl.BlockSpec((N_CHUNKS, ROWS // 64, 128),
                               lambda: (0, 0, 0)),
        out_shape=jax.ShapeDtypeStruct((N_CHUNKS, ROWS // 64, 128),
                                       jnp.float32),
    )(wp)


WT = 4096


def _tab_body(t0_ref, t1_ref, o0_ref, o1_ref):
    o0_ref[...] = jnp.transpose(t0_ref[...].reshape(128, WT))
    o1_ref[...] = jnp.transpose(t1_ref[...].reshape(128, WT))


def _tab_linearize(t0p, t1p):
    spec_i = pl.BlockSpec((N_CHUNKS, CHUNK, WT), lambda rb: (0, 0, rb))
    spec_o = pl.BlockSpec((WT, 128), lambda rb: (rb, 0))
    out_t = jax.ShapeDtypeStruct((ROWS, 128), jnp.float32)
    return pl.pallas_call(
        _tab_body,
        grid=(ROWS // WT,),
        in_specs=[spec_i, spec_i],
        out_specs=[spec_o, spec_o],
        out_shape=[out_t, out_t],
    )(t0p, t1p)


def _linearize(h0t, h1t, h2t):
    spec_i = pl.BlockSpec((N_CHUNKS, HB), lambda xb: (0, xb))
    spec_o = pl.BlockSpec((HB // 128, N_CHUNKS, 128), lambda xb: (xb, 0, 0))
    out_t = jax.ShapeDtypeStruct((PTILES, N_CHUNKS, 128), jnp.int32)
    return pl.pallas_call(
        _lin_body,
        grid=((VOCAB + HB - 1) // HB,),
        in_specs=[spec_i, spec_i, spec_i],
        out_specs=[spec_o, spec_o, spec_o],
        out_shape=[out_t, out_t, out_t],
    )(h0t, h1t, h2t)


def kernel(x, table0, table1, weights, h0, h1, h2):
    t0w, t1w = _tab_linearize(jnp.transpose(table0, (1, 2, 0)),
                              jnp.transpose(table1, (1, 2, 0)))
    t0 = t0w.reshape(ROWS * N_CHUNKS, CHUNK)
    t1 = t1w.reshape(ROWS * N_CHUNKS, CHUNK)
    w = _w_linearize(jnp.transpose(weights, (1, 2, 0))).reshape(
        ROWS * N_CHUNKS * 2)
    h0f, h1f, h2f = (h.reshape(PTILES * N_CHUNKS * 128)
                     for h in _linearize(jnp.transpose(h0), jnp.transpose(h1),
                                         jnp.transpose(h2)))
    i0, i1, iw0, iw1 = _call_a(x, h0f, h1f, h2f)
    return _call_b(t0, t1, w, i0, i1, iw0, iw1)


# speedup vs baseline: 1.0723x; 1.0084x over previous
import jax
import jax.numpy as jnp
from jax import lax
from jax.experimental import pallas as pl
from jax.experimental.pallas import tpu as pltpu
from jax.experimental.pallas import tpu_sc as plsc

VOCAB = 1000000
ROWS = 65536
N_CHUNKS = 8
CHUNK = 16
B = 16384

NC = 2
NS = 16
NW = NC * NS
TPW = B // NW
RPW = TPW * N_CHUNKS
RPS = TPW


def _idx_body(x_hbm, h0_hbm, h1_hbm, h2_hbm, i0_hbm, i1_hbm, iw0_hbm,
              iw1_hbm, x_v, xe_v, g0_v, g1_v, g2_v, sem):
    c = lax.axis_index("c")
    s = lax.axis_index("s")
    wid = s * NC + c
    tok_base = wid * TPW

    pltpu.sync_copy(x_hbm.at[pl.ds(tok_base, TPW)], x_v)

    def xe_body(i, _):
        cc = lax.shift_right_logical(i, 5)
        bo = jnp.bitwise_and(i, 31) * 16
        xv = x_v[pl.ds(bo, 16)]
        xe_v[pl.ds(i * 16, 16)] = (
            lax.shift_left(lax.shift_right_logical(xv, 7), 10)
            + cc * 128 + jnp.bitwise_and(xv, 127))
        return 0

    lax.fori_loop(0, RPW // 16, xe_body, 0)

    cp0 = pltpu.async_copy(h0_hbm.at[xe_v], g0_v, sem)
    cp1 = pltpu.async_copy(h1_hbm.at[xe_v], g1_v, sem)
    cp2 = pltpu.async_copy(h2_hbm.at[xe_v], g2_v, sem)
    cp0.wait()
    cp1.wait()
    cp2.wait()

    def idx_body(i, _):
        cc = lax.shift_right_logical(i, 5)
        sl = pl.ds(i * 16, 16)
        g0_v[sl] = g0_v[sl] * N_CHUNKS + cc
        g1_v[sl] = g1_v[sl] * N_CHUNKS + cc
        h2v = g2_v[sl]
        base = (cc * (2 * ROWS)
                + lax.shift_left(lax.shift_right_logical(h2v, 7), 8)
                + jnp.bitwise_and(h2v, 127))
        xe_v[sl] = base
        g2_v[sl] = base + 128
        return 0

    lax.fori_loop(0, RPW // 16, idx_body, 0)

    row_base = pl.ds(wid * RPW, RPW)
    pltpu.sync_copy(g0_v, i0_hbm.at[row_base])
    pltpu.sync_copy(g1_v, i1_hbm.at[row_base])
    pltpu.sync_copy(xe_v, iw0_hbm.at[row_base])
    pltpu.sync_copy(g2_v, iw1_hbm.at[row_base])


def _gather_body(t0_hbm, t1_hbm, w_hbm, i0_hbm, i1_hbm, iw0_hbm, iw1_hbm,
                 out_hbm, g0_v, g1_v, iw0_v, iw1_v, w0_v, w1_v, p0a, p1a,
                 p0b, p1b, out_v, sem, sem_b):
    c = lax.axis_index("c")
    s = lax.axis_index("s")
    wid = s * NC + c
    tok_base = wid * TPW
    row_base = pl.ds(wid * RPW, RPW)

    cis = [pltpu.async_copy(i0_hbm.at[row_base], g0_v, sem),
           pltpu.async_copy(i1_hbm.at[row_base], g1_v, sem),
           pltpu.async_copy(iw0_hbm.at[row_base], iw0_v, sem),
           pltpu.async_copy(iw1_hbm.at[row_base], iw1_v, sem)]
    for ci in cis:
        ci.wait()

    cpw0 = pltpu.async_copy(w_hbm.at[iw0_v], w0_v, sem)
    cpw1 = pltpu.async_copy(w_hbm.at[iw1_v], w1_v, sem)

    bufs = ((p0a, p1a), (p0b, p1b))
    sems = (sem, sem_b)

    def fire(sb):
        bb = sb & 1
        sl = pl.ds(sb * RPS, RPS)
        return (pltpu.async_copy(t0_hbm.at[g0_v.at[sl]], bufs[bb][0],
                                 sems[bb]),
                pltpu.async_copy(t1_hbm.at[g1_v.at[sl]], bufs[bb][1],
                                 sems[bb]))

    cur = fire(0)
    for sb in range(N_CHUNKS):
        nxt = fire(sb + 1) if sb + 1 < N_CHUNKS else None
        if sb == 0:
            cpw0.wait()
            cpw1.wait()
        cur[0].wait()
        cur[1].wait()
        p0_v, p1_v = bufs[sb & 1]

        def row_body(m, _):
            qv = jnp.full((16,), sb * RPS + m, jnp.int32)
            w0 = plsc.load_gather(w0_v, [qv])
            w1 = plsc.load_gather(w1_v, [qv])
            out_v[m, pl.ds(sb * CHUNK, CHUNK)] = (
                p0_v[m, :] * w0 + p1_v[m, :] * w1)
            return 0

        lax.fori_loop(0, RPS, row_body, 0)
        cur = nxt

    pltpu.sync_copy(out_v, out_hbm.at[pl.ds(tok_base, TPW), :])


@jax.jit
def _call_a(x, h0f, h1f, h2f):
    mesh = plsc.VectorSubcoreMesh(core_axis_name="c", subcore_axis_name="s")
    it = jax.ShapeDtypeStruct((B * N_CHUNKS,), jnp.int32)
    run = pl.kernel(
        _idx_body,
        out_type=(it, it, it, it),
        mesh=mesh,
        compiler_params=pltpu.CompilerParams(use_tc_tiling_on_sc=False,
                                             needs_layout_passes=False),
        scratch_types=[
            pltpu.VMEM((TPW,), jnp.int32),
            pltpu.VMEM((RPW,), jnp.int32),
            pltpu.VMEM((RPW,), jnp.int32),
            pltpu.VMEM((RPW,), jnp.int32),
            pltpu.VMEM((RPW,), jnp.int32),
            pltpu.SemaphoreType.DMA,
        ],
    )
    return run(x, h0f, h1f, h2f)


@jax.jit
def _call_b(t0, t1, w, i0, i1, iw0, iw1):
    mesh = plsc.VectorSubcoreMesh(core_axis_name="c", subcore_axis_name="s")
    run = pl.kernel(
        _gather_body,
        out_type=jax.ShapeDtypeStruct((B, N_CHUNKS * CHUNK), jnp.float32),
        mesh=mesh,
        compiler_params=pltpu.CompilerParams(use_tc_tiling_on_sc=False,
                                             needs_layout_passes=False),
        scratch_types=[
            pltpu.VMEM((RPW,), jnp.int32),
            pltpu.VMEM((RPW,), jnp.int32),
            pltpu.VMEM((RPW,), jnp.int32),
            pltpu.VMEM((RPW,), jnp.int32),
            pltpu.VMEM((RPW,), jnp.float32),
            pltpu.VMEM((RPW,), jnp.float32),
            pltpu.VMEM((RPS, CHUNK), jnp.float32),
            pltpu.VMEM((RPS, CHUNK), jnp.float32),
            pltpu.VMEM((RPS, CHUNK), jnp.float32),
            pltpu.VMEM((RPS, CHUNK), jnp.float32),
            pltpu.VMEM((TPW, 128), jnp.float32),
            pltpu.SemaphoreType.DMA,
            pltpu.SemaphoreType.DMA,
        ],
    )
    return run(t0, t1, w, i0, i1, iw0, iw1)


HB = 65536
TILES = (VOCAB + 127) // 128
PTILES = 7816
HPAD = PTILES * 128


def _lin_body(i0_ref, i1_ref, i2_ref, o0_ref, o1_ref, o2_ref):
    for i_ref, o_ref in ((i0_ref, o0_ref), (i1_ref, o1_ref),
                         (i2_ref, o2_ref)):
        o_ref[...] = jnp.transpose(
            i_ref[...].reshape(N_CHUNKS, HB // 128, 128), (1, 0, 2))


def _w_body(w_ref, o_ref):
    t = w_ref[...].reshape(N_CHUNKS, 2, ROWS // 128, 128)
    o_ref[...] = jnp.transpose(t, (0, 2, 1, 3)).reshape(
        N_CHUNKS, ROWS // 64, 128)


def _w_linearize(wp):
    return pl.pallas_call(
        _w_body,
        in_specs=[pl.BlockSpec((N_CHUNKS, 2, ROWS), lambda: (0, 0, 0))],
        out_specs=pl.BlockSpec((N_CHUNKS, ROWS // 64, 128),
                               lambda: (0, 0, 0)),
        out_shape=jax.ShapeDtypeStruct((N_CHUNKS, ROWS // 64, 128),
                                       jnp.float32),
    )(wp)


WT = 4096


def _tab_body(t0_ref, t1_ref, o0_ref, o1_ref):
    o0_ref[...] = jnp.transpose(t0_ref[...].reshape(128, WT))
    o1_ref[...] = jnp.transpose(t1_ref[...].reshape(128, WT))


def _tab_linearize(t0p, t1p):
    spec_i = pl.BlockSpec((N_CHUNKS, CHUNK, WT), lambda rb: (0, 0, rb))
    spec_o = pl.BlockSpec((WT, 128), lambda rb: (rb, 0))
    out_t = jax.ShapeDtypeStruct((ROWS, 128), jnp.float32)
    return pl.pallas_call(
        _tab_body,
        grid=(ROWS // WT,),
        in_specs=[spec_i, spec_i],
        out_specs=[spec_o, spec_o],
        out_shape=[out_t, out_t],
    )(t0p, t1p)


def _linearize(h0t, h1t, h2t):
    spec_i = pl.BlockSpec((N_CHUNKS, HB), lambda xb: (0, xb))
    spec_o = pl.BlockSpec((HB // 128, N_CHUNKS, 128), lambda xb: (xb, 0, 0))
    out_t = jax.ShapeDtypeStruct((PTILES, N_CHUNKS, 128), jnp.int32)
    return pl.pallas_call(
        _lin_body,
        grid=((VOCAB + HB - 1) // HB,),
        in_specs=[spec_i, spec_i, spec_i],
        out_specs=[spec_o, spec_o, spec_o],
        out_shape=[out_t, out_t, out_t],
    )(h0t, h1t, h2t)


def kernel(x, table0, table1, weights, h0, h1, h2):
    t0w, t1w = _tab_linearize(jnp.transpose(table0, (1, 2, 0)),
                              jnp.transpose(table1, (1, 2, 0)))
    t0 = t0w.reshape(ROWS * N_CHUNKS, CHUNK)
    t1 = t1w.reshape(ROWS * N_CHUNKS, CHUNK)
    w = _w_linearize(jnp.transpose(weights, (1, 2, 0))).reshape(
        ROWS * N_CHUNKS * 2)
    h0f, h1f, h2f = (h.reshape(PTILES * N_CHUNKS * 128)
                     for h in _linearize(jnp.transpose(h0), jnp.transpose(h1),
                                         jnp.transpose(h2)))
    i0, i1, iw0, iw1 = _call_a(x, h0f, h1f, h2f)
    return _call_b(t0, t1, w, i0, i1, iw0, iw1)
